# baseline (speedup 1.0000x reference)
import functools

import jax
import jax.numpy as jnp
from jax import lax
from jax.experimental import pallas as pl
from jax.experimental.pallas import tpu as pltpu
from jax.experimental.pallas import tpu_sc as plsc

VOCAB = 100000
EMBED = 64
N_ROWS = 16384
N_COLS = 50

NUM_CORES = 2
NUM_SUBCORES = 16
NW = NUM_CORES * NUM_SUBCORES
R_PER_W = N_ROWS // NW
K = 8
NPAIRS = R_PER_W // (2 * K)


def _make_gather():
    mesh = plsc.VectorSubcoreMesh(core_axis_name="c", subcore_axis_name="s")

    @functools.partial(
        pl.kernel,
        mesh=mesh,
        out_type=jax.ShapeDtypeStruct((N_ROWS, N_COLS, EMBED), jnp.float32),
        scratch_types=[
            pltpu.VMEM((R_PER_W, N_COLS), jnp.int32),
            pltpu.VMEM((K, N_COLS, EMBED), jnp.float32),
            pltpu.VMEM((K, N_COLS, EMBED), jnp.float32),
            pltpu.SemaphoreType.DMA,
            pltpu.SemaphoreType.DMA,
            pltpu.SemaphoreType.DMA,
            pltpu.SemaphoreType.DMA,
        ],
        compiler_params=pltpu.CompilerParams(use_tc_tiling_on_sc=False),
    )
    def gather_kernel(idx_hbm, table_hbm, out_hbm, idx_v, rows_a, rows_b,
                      gsem_a, gsem_b, ssem_a, ssem_b):
        wid = lax.axis_index("s") * NUM_CORES + lax.axis_index("c")
        base = wid * R_PER_W
        pltpu.sync_copy(idx_hbm.at[pl.ds(base, R_PER_W)], idx_v)

        def fg(rows, gsem, t):
            for b in range(K):
                pltpu.async_copy(
                    table_hbm.at[idx_v.at[t * K + b]], rows.at[b], gsem)

        def dg(rows, gsem):
            for b in range(K):
                pltpu.make_async_copy(
                    table_hbm.at[pl.ds(0, N_COLS)], rows.at[b], gsem).wait()

        def fs(rows, ssem, t):
            for b in range(K):
                pltpu.async_copy(
                    rows.at[b], out_hbm.at[base + t * K + b], ssem)

        def ds(rows, ssem):
            for b in range(K):
                pltpu.make_async_copy(
                    rows.at[b], out_hbm.at[0], ssem).wait()

        fg(rows_a, gsem_a, 0)
        dg(rows_a, gsem_a)
        fg(rows_b, gsem_b, 1)
        fs(rows_a, ssem_a, 0)
        dg(rows_b, gsem_b)
        ds(rows_a, ssem_a)
        fg(rows_a, gsem_a, 2)
        fs(rows_b, ssem_b, 1)

        def body(p, carry):
            dg(rows_a, gsem_a)
            ds(rows_b, ssem_b)
            fg(rows_b, gsem_b, 2 * p + 1)
            fs(rows_a, ssem_a, 2 * p)
            dg(rows_b, gsem_b)
            ds(rows_a, ssem_a)
            fg(rows_a, gsem_a, 2 * p + 2)
            fs(rows_b, ssem_b, 2 * p + 1)
            return carry

        lax.fori_loop(1, NPAIRS - 1, body, 0)

        t1 = 2 * NPAIRS - 1
        dg(rows_a, gsem_a)
        ds(rows_b, ssem_b)
        fg(rows_b, gsem_b, t1)
        fs(rows_a, ssem_a, t1 - 1)
        dg(rows_b, gsem_b)
        ds(rows_a, ssem_a)
        fs(rows_b, ssem_b, t1)
        ds(rows_b, ssem_b)

    return gather_kernel


_gather = _make_gather()


def kernel(indices, table):
    return _gather(indices.astype(jnp.int32), table)
